# Initial kernel scaffold; baseline (speedup 1.0000x reference)
import functools

import jax
import jax.numpy as jnp
from jax import lax
from jax.experimental import pallas as pl
from jax.experimental.pallas import tpu as pltpu
from jax.experimental.pallas import tpu_sc as plsc

_NC = 2
_NS = 16
_CHUNK = 80
_EPS = 1e-5


def _silu(v):
    return v * jax.nn.sigmoid(v)


def _gather_body(nchunk, ept, x_hbm, idx_hbm, out_hbm, idx_v, buf_a, buf_b,
                 sem_a, sem_b):
    cid = lax.axis_index("c")
    sid = lax.axis_index("s")
    wid = cid * _NS + sid
    pltpu.sync_copy(idx_hbm.at[wid], idx_v)
    base = wid * ept

    def body(c, carry):
        a = pltpu.async_copy(x_hbm.at[idx_v.at[0, c]], buf_a, sem_a)
        b = pltpu.async_copy(x_hbm.at[idx_v.at[1, c]], buf_b, sem_b)
        a.wait()
        pltpu.sync_copy(buf_a, out_hbm.at[0, pl.ds(base + c * _CHUNK, _CHUNK)])
        b.wait()
        pltpu.sync_copy(buf_b, out_hbm.at[1, pl.ds(base + c * _CHUNK, _CHUNK)])
        return carry

    lax.fori_loop(0, nchunk, body, 0)


def _mlp_body(fc, neb, xi_ref, xj_ref, ea_ref, w1_ref, w2_ref, w3_ref, w4_ref,
              b1_ref, b2_ref, b3_ref, b4_ref, z_ref, m_ref, zs_ref, zq_ref,
              acc_s, acc_q):
    i = pl.program_id(0)
    xi = xi_ref[0]
    xj = xj_ref[0]
    ea = ea_ref[...]
    w1 = w1_ref[...]
    w3 = w3_ref[...]

    a1 = (jnp.dot(xi, w1[:fc], preferred_element_type=jnp.float32)
          + jnp.dot(xj, w1[fc:2 * fc], preferred_element_type=jnp.float32)
          + jnp.dot(ea, w1[2 * fc:], preferred_element_type=jnp.float32)
          + b1_ref[...])
    z = jnp.dot(_silu(a1), w2_ref[...], preferred_element_type=jnp.float32) \
        + b2_ref[...]
    a3 = (jnp.dot(xi, w3[:fc], preferred_element_type=jnp.float32)
          + jnp.dot(xj, w3[fc:2 * fc], preferred_element_type=jnp.float32)
          + jnp.dot(ea, w3[2 * fc:], preferred_element_type=jnp.float32)
          + b3_ref[...])
    m = jnp.dot(_silu(a3), w4_ref[...], preferred_element_type=jnp.float32) \
        + b4_ref[...]

    z_ref[...] = z
    m_ref[...] = m

    @pl.when(i == 0)
    def _():
        acc_s[...] = jnp.zeros_like(acc_s)
        acc_q[...] = jnp.zeros_like(acc_q)

    acc_s[...] += jnp.sum(z, axis=0, keepdims=True)
    acc_q[...] += jnp.sum(z * z, axis=0, keepdims=True)

    @pl.when(i == neb - 1)
    def _():
        zs_ref[...] = acc_s[...]
        zq_ref[...] = acc_q[...]


def _score_body(e_f, z_ref, m_ref, zs_ref, zq_ref, g_ref, b_ref, msg_ref):
    mu = zs_ref[...] / e_f
    var = zq_ref[...] / e_f - mu * mu
    rstd = lax.rsqrt(var + _EPS)
    zn = (z_ref[...] - mu) * (rstd * g_ref[...]) + b_ref[...]
    score = jax.nn.sigmoid(zn)
    msg_ref[...] = score * m_ref[...]


def _scatter_body(n, fc, nchunk, ept, rch, msg_hbm, idx_hbm, out_hbm,
                  shared, vbuf, mbuf, idxs):
    cid = lax.axis_index("c")
    sid = lax.axis_index("s")
    wid = cid * _NS + sid
    rows_per_sub = n // _NS
    nrc = rows_per_sub // rch

    zero16 = jnp.zeros((16,), jnp.float32)

    def zb(k, carry):
        vbuf[k // 8, pl.ds((k % 8) * 16, 16)] = zero16
        return carry

    lax.fori_loop(0, rch * (fc // 16), zb, 0)

    def zcopy(k, carry):
        pltpu.sync_copy(vbuf,
                        shared.at[pl.ds(sid * rows_per_sub + k * rch, rch)])
        return carry

    lax.fori_loop(0, nrc, zcopy, 0)
    plsc.subcore_barrier()

    pltpu.sync_copy(idx_hbm.at[wid], idxs)
    ebase = wid * ept

    def sb(c, carry):
        pltpu.sync_copy(msg_hbm.at[pl.ds(ebase + c * _CHUNK, _CHUNK)], mbuf)
        pltpu.sync_copy(mbuf, shared.at[idxs.at[c]], add=True)
        return carry

    lax.fori_loop(0, nchunk, sb, 0)
    plsc.subcore_barrier()

    def ob(k, carry):
        r0 = sid * rows_per_sub + k * rch
        pltpu.sync_copy(shared.at[pl.ds(r0, rch)], vbuf)
        pltpu.sync_copy(vbuf, out_hbm.at[cid, pl.ds(r0, rch)])
        return carry

    lax.fori_loop(0, nrc, ob, 0)


def _final_body(n_f, p0_ref, p1_ref, x_ref, g_ref, b_ref, y_ref, acc_s, acc_q):
    ph = pl.program_id(0)
    i = pl.program_id(1)
    o = p0_ref[0] + p1_ref[0]

    @pl.when(ph == 0)
    def _():
        @pl.when(i == 0)
        def _():
            acc_s[...] = jnp.zeros_like(acc_s)
            acc_q[...] = jnp.zeros_like(acc_q)

        acc_s[...] += jnp.sum(o, axis=0, keepdims=True)
        acc_q[...] += jnp.sum(o * o, axis=0, keepdims=True)

    @pl.when(ph == 1)
    def _():
        mu = acc_s[...] / n_f
        var = acc_q[...] / n_f - mu * mu
        rstd = lax.rsqrt(var + _EPS)
        y_ref[...] = jnp.maximum(
            x_ref[...] + (o - mu) * (rstd * g_ref[...]) + b_ref[...], 0.0)


def kernel(x, edge_index, edge_attr, W1, b1, W2, b2, W3, b3, W4, b4,
           g_int, b_int, g_bn, b_bn):
    f32 = jnp.float32
    n, fc = x.shape
    e = edge_index.shape[1]
    nw = _NC * _NS
    ept = e // nw
    nchunk = ept // _CHUNK
    eb = 2560
    neb = e // eb
    nb = 2000
    rch = 125

    src = edge_index[0]
    dst = edge_index[1]
    idx_g = jnp.stack([src.reshape(nw, nchunk, _CHUNK),
                       dst.reshape(nw, nchunk, _CHUNK)], axis=1)
    dst3 = dst.reshape(nw, nchunk, _CHUNK)

    mesh = plsc.VectorSubcoreMesh(core_axis_name="c", subcore_axis_name="s")

    gathered = pl.kernel(
        functools.partial(_gather_body, nchunk, ept),
        out_type=jax.ShapeDtypeStruct((2, e, fc), f32),
        mesh=mesh,
        scratch_types=[
            pltpu.VMEM((2, nchunk, _CHUNK), jnp.int32),
            pltpu.VMEM((_CHUNK, fc), f32),
            pltpu.VMEM((_CHUNK, fc), f32),
            pltpu.SemaphoreType.DMA,
            pltpu.SemaphoreType.DMA,
        ],
    )(x, idx_g)

    z, m, zsum, zsq = pl.pallas_call(
        functools.partial(_mlp_body, fc, neb),
        grid=(neb,),
        in_specs=[
            pl.BlockSpec((1, eb, fc), lambda i: (1, i, 0)),
            pl.BlockSpec((1, eb, fc), lambda i: (0, i, 0)),
            pl.BlockSpec((eb, fc), lambda i: (i, 0)),
            pl.BlockSpec((3 * fc, fc), lambda i: (0, 0)),
            pl.BlockSpec((fc, fc), lambda i: (0, 0)),
            pl.BlockSpec((3 * fc, fc), lambda i: (0, 0)),
            pl.BlockSpec((fc, fc), lambda i: (0, 0)),
            pl.BlockSpec((1, fc), lambda i: (0, 0)),
            pl.BlockSpec((1, fc), lambda i: (0, 0)),
            pl.BlockSpec((1, fc), lambda i: (0, 0)),
            pl.BlockSpec((1, fc), lambda i: (0, 0)),
        ],
        out_specs=[
            pl.BlockSpec((eb, fc), lambda i: (i, 0)),
            pl.BlockSpec((eb, fc), lambda i: (i, 0)),
            pl.BlockSpec((1, fc), lambda i: (0, 0)),
            pl.BlockSpec((1, fc), lambda i: (0, 0)),
        ],
        out_shape=[
            jax.ShapeDtypeStruct((e, fc), f32),
            jax.ShapeDtypeStruct((e, fc), f32),
            jax.ShapeDtypeStruct((1, fc), f32),
            jax.ShapeDtypeStruct((1, fc), f32),
        ],
        scratch_shapes=[pltpu.VMEM((1, fc), f32), pltpu.VMEM((1, fc), f32)],
    )(gathered, gathered, edge_attr, W1, W2, W3, W4,
      b1.reshape(1, fc), b2.reshape(1, fc), b3.reshape(1, fc),
      b4.reshape(1, fc))

    msg = pl.pallas_call(
        functools.partial(_score_body, float(e)),
        grid=(neb,),
        in_specs=[
            pl.BlockSpec((eb, fc), lambda i: (i, 0)),
            pl.BlockSpec((eb, fc), lambda i: (i, 0)),
            pl.BlockSpec((1, fc), lambda i: (0, 0)),
            pl.BlockSpec((1, fc), lambda i: (0, 0)),
            pl.BlockSpec((1, fc), lambda i: (0, 0)),
            pl.BlockSpec((1, fc), lambda i: (0, 0)),
        ],
        out_specs=pl.BlockSpec((eb, fc), lambda i: (i, 0)),
        out_shape=jax.ShapeDtypeStruct((e, fc), f32),
    )(z, m, zsum, zsq, g_int.reshape(1, fc), b_int.reshape(1, fc))

    partials = pl.kernel(
        functools.partial(_scatter_body, n, fc, nchunk, ept, rch),
        out_type=jax.ShapeDtypeStruct((_NC, n, fc), f32),
        mesh=mesh,
        scratch_types=[
            pltpu.VMEM_SHARED((n, fc), f32),
            pltpu.VMEM((rch, fc), f32),
            pltpu.VMEM((_CHUNK, fc), f32),
            pltpu.VMEM((nchunk, _CHUNK), jnp.int32),
        ],
    )(msg, dst3)

    y = pl.pallas_call(
        functools.partial(_final_body, float(n)),
        grid=(2, n // nb),
        in_specs=[
            pl.BlockSpec((1, nb, fc), lambda p, i: (0, i, 0)),
            pl.BlockSpec((1, nb, fc), lambda p, i: (1, i, 0)),
            pl.BlockSpec((nb, fc), lambda p, i: (i, 0)),
            pl.BlockSpec((1, fc), lambda p, i: (0, 0)),
            pl.BlockSpec((1, fc), lambda p, i: (0, 0)),
        ],
        out_specs=pl.BlockSpec((nb, fc), lambda p, i: (i, 0)),
        out_shape=jax.ShapeDtypeStruct((n, fc), f32),
        scratch_shapes=[pltpu.VMEM((1, fc), f32), pltpu.VMEM((1, fc), f32)],
    )(partials, partials, x, g_bn.reshape(1, fc), b_bn.reshape(1, fc))

    return y


# speedup vs baseline: 2.8302x; 2.8302x over previous
import functools

import jax
import jax.numpy as jnp
from jax import lax
from jax.experimental import pallas as pl
from jax.experimental.pallas import tpu as pltpu
from jax.experimental.pallas import tpu_sc as plsc

_NC = 2
_NS = 16
_CHUNK = 80
_EPS = 1e-5


def _silu(v):
    return v * jax.nn.sigmoid(v)


def _gather_body(nchunk, ept, x_hbm, idx_hbm, out_hbm, idx_v, buf_a, buf_b,
                 sem_a, sem_b):
    cid = lax.axis_index("c")
    sid = lax.axis_index("s")
    wid = cid * _NS + sid
    pltpu.sync_copy(idx_hbm.at[wid], idx_v)
    base = wid * ept

    def body(c, carry):
        a = pltpu.async_copy(x_hbm.at[idx_v.at[0, c]], buf_a, sem_a)
        b = pltpu.async_copy(x_hbm.at[idx_v.at[1, c]], buf_b, sem_b)
        a.wait()
        pltpu.sync_copy(buf_a, out_hbm.at[0, pl.ds(base + c * _CHUNK, _CHUNK)])
        b.wait()
        pltpu.sync_copy(buf_b, out_hbm.at[1, pl.ds(base + c * _CHUNK, _CHUNK)])
        return carry

    lax.fori_loop(0, nchunk, body, 0)


def _mlp_body(fc, neb, xi_ref, xj_ref, ea_ref, w1_ref, w2_ref, w3_ref, w4_ref,
              b1_ref, b2_ref, b3_ref, b4_ref, z_ref, m_ref, zs_ref, zq_ref,
              acc_s, acc_q):
    i = pl.program_id(0)
    xi = xi_ref[0]
    xj = xj_ref[0]
    ea = ea_ref[...]
    w1 = w1_ref[...]
    w3 = w3_ref[...]

    a1 = (jnp.dot(xi, w1[:fc], preferred_element_type=jnp.float32)
          + jnp.dot(xj, w1[fc:2 * fc], preferred_element_type=jnp.float32)
          + jnp.dot(ea, w1[2 * fc:], preferred_element_type=jnp.float32)
          + b1_ref[...])
    z = jnp.dot(_silu(a1), w2_ref[...], preferred_element_type=jnp.float32) \
        + b2_ref[...]
    a3 = (jnp.dot(xi, w3[:fc], preferred_element_type=jnp.float32)
          + jnp.dot(xj, w3[fc:2 * fc], preferred_element_type=jnp.float32)
          + jnp.dot(ea, w3[2 * fc:], preferred_element_type=jnp.float32)
          + b3_ref[...])
    m = jnp.dot(_silu(a3), w4_ref[...], preferred_element_type=jnp.float32) \
        + b4_ref[...]

    z_ref[...] = z
    m_ref[...] = m

    @pl.when(i == 0)
    def _():
        acc_s[...] = jnp.zeros_like(acc_s)
        acc_q[...] = jnp.zeros_like(acc_q)

    acc_s[...] += jnp.sum(z, axis=0, keepdims=True)
    acc_q[...] += jnp.sum(z * z, axis=0, keepdims=True)

    @pl.when(i == neb - 1)
    def _():
        zs_ref[...] = acc_s[...]
        zq_ref[...] = acc_q[...]


def _score_body(e_f, z_ref, m_ref, zs_ref, zq_ref, g_ref, b_ref, msg_ref):
    mu = zs_ref[...] / e_f
    var = zq_ref[...] / e_f - mu * mu
    rstd = lax.rsqrt(var + _EPS)
    zn = (z_ref[...] - mu) * (rstd * g_ref[...]) + b_ref[...]
    score = jax.nn.sigmoid(zn)
    msg_ref[...] = score * m_ref[...]


def _scatter_body(n, fc, nchunk, ept, rch, msg_hbm, idx_hbm, out_hbm,
                  shared, vbuf, mbuf, idxs):
    cid = lax.axis_index("c")
    sid = lax.axis_index("s")
    wid = cid * _NS + sid
    nzc = n // rch
    iters = (nzc + _NS - 1) // _NS

    zero16 = jnp.zeros((16,), jnp.float32)
    lanes = fc // 16

    def zb(k, carry):
        vbuf[k // lanes, pl.ds((k % lanes) * 16, 16)] = zero16
        return carry

    lax.fori_loop(0, rch * lanes, zb, 0)

    def zcopy(k, carry):
        c = k * _NS + sid

        @pl.when(c < nzc)
        def _():
            pltpu.sync_copy(vbuf, shared.at[pl.ds(c * rch, rch)])

        return carry

    lax.fori_loop(0, iters, zcopy, 0)
    plsc.subcore_barrier()

    pltpu.sync_copy(idx_hbm.at[wid], idxs)
    ebase = wid * ept

    def sb(c, carry):
        pltpu.sync_copy(msg_hbm.at[pl.ds(ebase + c * _CHUNK, _CHUNK)], mbuf)
        pltpu.sync_copy(mbuf, shared.at[idxs.at[c]], add=True)
        return carry

    lax.fori_loop(0, nchunk, sb, 0)
    plsc.subcore_barrier()

    def ob(k, carry):
        c = k * _NS + sid

        @pl.when(c < nzc)
        def _():
            pltpu.sync_copy(shared.at[pl.ds(c * rch, rch)], vbuf)
            pltpu.sync_copy(vbuf, out_hbm.at[cid, pl.ds(c * rch, rch)])

        return carry

    lax.fori_loop(0, iters, ob, 0)


def _final_body(n_f, p0_ref, p1_ref, x_ref, g_ref, b_ref, y_ref, acc_s, acc_q):
    ph = pl.program_id(0)
    i = pl.program_id(1)
    o = p0_ref[0] + p1_ref[0]

    @pl.when(ph == 0)
    def _():
        @pl.when(i == 0)
        def _():
            acc_s[...] = jnp.zeros_like(acc_s)
            acc_q[...] = jnp.zeros_like(acc_q)

        acc_s[...] += jnp.sum(o, axis=0, keepdims=True)
        acc_q[...] += jnp.sum(o * o, axis=0, keepdims=True)

    @pl.when(ph == 1)
    def _():
        mu = acc_s[...] / n_f
        var = acc_q[...] / n_f - mu * mu
        rstd = lax.rsqrt(var + _EPS)
        y_ref[...] = jnp.maximum(
            x_ref[...] + (o - mu) * (rstd * g_ref[...]) + b_ref[...], 0.0)


def kernel(x, edge_index, edge_attr, W1, b1, W2, b2, W3, b3, W4, b4,
           g_int, b_int, g_bn, b_bn):
    f32 = jnp.float32
    n, fc = x.shape
    e = edge_index.shape[1]
    nw = _NC * _NS
    ept = e // nw
    nchunk = ept // _CHUNK
    eb = 2560
    neb = e // eb
    nb = 2000
    rch = 80

    src = edge_index[0]
    dst = edge_index[1]
    idx_g = jnp.stack([src.reshape(nw, nchunk, _CHUNK),
                       dst.reshape(nw, nchunk, _CHUNK)], axis=1)
    dst3 = dst.reshape(nw, nchunk, _CHUNK)

    mesh = plsc.VectorSubcoreMesh(core_axis_name="c", subcore_axis_name="s")

    gathered = pl.kernel(
        functools.partial(_gather_body, nchunk, ept),
        out_type=jax.ShapeDtypeStruct((2, e, fc), f32),
        mesh=mesh,
        scratch_types=[
            pltpu.VMEM((2, nchunk, _CHUNK), jnp.int32),
            pltpu.VMEM((_CHUNK, fc), f32),
            pltpu.VMEM((_CHUNK, fc), f32),
            pltpu.SemaphoreType.DMA,
            pltpu.SemaphoreType.DMA,
        ],
    )(x, idx_g)

    z, m, zsum, zsq = pl.pallas_call(
        functools.partial(_mlp_body, fc, neb),
        grid=(neb,),
        in_specs=[
            pl.BlockSpec((1, eb, fc), lambda i: (1, i, 0)),
            pl.BlockSpec((1, eb, fc), lambda i: (0, i, 0)),
            pl.BlockSpec((eb, fc), lambda i: (i, 0)),
            pl.BlockSpec((3 * fc, fc), lambda i: (0, 0)),
            pl.BlockSpec((fc, fc), lambda i: (0, 0)),
            pl.BlockSpec((3 * fc, fc), lambda i: (0, 0)),
            pl.BlockSpec((fc, fc), lambda i: (0, 0)),
            pl.BlockSpec((1, fc), lambda i: (0, 0)),
            pl.BlockSpec((1, fc), lambda i: (0, 0)),
            pl.BlockSpec((1, fc), lambda i: (0, 0)),
            pl.BlockSpec((1, fc), lambda i: (0, 0)),
        ],
        out_specs=[
            pl.BlockSpec((eb, fc), lambda i: (i, 0)),
            pl.BlockSpec((eb, fc), lambda i: (i, 0)),
            pl.BlockSpec((1, fc), lambda i: (0, 0)),
            pl.BlockSpec((1, fc), lambda i: (0, 0)),
        ],
        out_shape=[
            jax.ShapeDtypeStruct((e, fc), f32),
            jax.ShapeDtypeStruct((e, fc), f32),
            jax.ShapeDtypeStruct((1, fc), f32),
            jax.ShapeDtypeStruct((1, fc), f32),
        ],
        scratch_shapes=[pltpu.VMEM((1, fc), f32), pltpu.VMEM((1, fc), f32)],
    )(gathered, gathered, edge_attr, W1, W2, W3, W4,
      b1.reshape(1, fc), b2.reshape(1, fc), b3.reshape(1, fc),
      b4.reshape(1, fc))

    msg = pl.pallas_call(
        functools.partial(_score_body, float(e)),
        grid=(neb,),
        in_specs=[
            pl.BlockSpec((eb, fc), lambda i: (i, 0)),
            pl.BlockSpec((eb, fc), lambda i: (i, 0)),
            pl.BlockSpec((1, fc), lambda i: (0, 0)),
            pl.BlockSpec((1, fc), lambda i: (0, 0)),
            pl.BlockSpec((1, fc), lambda i: (0, 0)),
            pl.BlockSpec((1, fc), lambda i: (0, 0)),
        ],
        out_specs=pl.BlockSpec((eb, fc), lambda i: (i, 0)),
        out_shape=jax.ShapeDtypeStruct((e, fc), f32),
    )(z, m, zsum, zsq, g_int.reshape(1, fc), b_int.reshape(1, fc))

    partials = pl.kernel(
        functools.partial(_scatter_body, n, fc, nchunk, ept, rch),
        out_type=jax.ShapeDtypeStruct((_NC, n, fc), f32),
        mesh=mesh,
        scratch_types=[
            pltpu.VMEM_SHARED((n, fc), f32),
            pltpu.VMEM((rch, fc), f32),
            pltpu.VMEM((_CHUNK, fc), f32),
            pltpu.VMEM((nchunk, _CHUNK), jnp.int32),
        ],
    )(msg, dst3)

    y = pl.pallas_call(
        functools.partial(_final_body, float(n)),
        grid=(2, n // nb),
        in_specs=[
            pl.BlockSpec((1, nb, fc), lambda p, i: (0, i, 0)),
            pl.BlockSpec((1, nb, fc), lambda p, i: (1, i, 0)),
            pl.BlockSpec((nb, fc), lambda p, i: (i, 0)),
            pl.BlockSpec((1, fc), lambda p, i: (0, 0)),
            pl.BlockSpec((1, fc), lambda p, i: (0, 0)),
        ],
        out_specs=pl.BlockSpec((nb, fc), lambda p, i: (i, 0)),
        out_shape=jax.ShapeDtypeStruct((n, fc), f32),
        scratch_shapes=[pltpu.VMEM((1, fc), f32), pltpu.VMEM((1, fc), f32)],
    )(partials, partials, x, g_bn.reshape(1, fc), b_bn.reshape(1, fc))

    return y
